# Initial kernel scaffold; baseline (speedup 1.0000x reference)
import jax
import jax.numpy as jnp
from jax.experimental import pallas as pl


def kernel(q, k, v):
    raise NotImplementedError("write your pallas kernel here")


# speedup vs baseline: 239.1008x; 239.1008x over previous
import functools
import math

import jax
import jax.numpy as jnp
from jax import lax
from jax.experimental import pallas as pl
from jax.experimental.pallas import tpu as pltpu
from jax.experimental.pallas import tpu_sc as plsc

B, Lq, L, D = 2, 2048, 2048, 64
RUNS, STEPS = 4, 32
BETA = 1.0
SCALE = 1.0 / math.sqrt(D)
NQ = B * Lq
NCH = NQ * RUNS
NW = 32
QPW = NQ // NW
QPG = 16
NG = QPW // QPG
GROUP = RUNS * STEPS * QPG
ROWS = QPG * L


def _qk_body(q_ref, kt_ref, s_ref):
    s_ref[0] = lax.dot_general(
        q_ref[0], kt_ref[0], (((1,), (0,)), ((), ())),
        precision=lax.Precision.HIGHEST,
        preferred_element_type=jnp.float32) * SCALE


def _wv_body(w_ref, v_ref, o_ref):
    o_ref[0] = lax.dot_general(
        w_ref[0], v_ref[0], (((1,), (0,)), ((), ())),
        precision=lax.Precision.HIGHEST,
        preferred_element_type=jnp.float32)


def _sc_body(s_hbm, ix_hbm, th_hbm, w_hbm, sg_v, mg_v, wg_v, ixb, thb, sgb):
    c = lax.axis_index("c")
    s = lax.axis_index("s")
    w = s * 2 + c
    zero16 = jnp.zeros((16,), jnp.float32)

    def zinit(i, carry):
        mg_v[pl.ds(i * 16, 16)] = zero16
        wg_v[pl.ds(i * 16, 16)] = zero16
        return carry

    lax.fori_loop(0, ROWS // 16, zinit, 0)

    def group_body(g, carry):
        soff = (w * NG + g) * ROWS
        goff = (w * NG + g) * GROUP
        pltpu.sync_copy(s_hbm.at[pl.ds(soff, ROWS)], sg_v)
        pltpu.sync_copy(ix_hbm.at[pl.ds(goff, GROUP)], ixb)
        pltpu.sync_copy(th_hbm.at[pl.ds(goff, GROUP)], thb)
        for r in range(RUNS):
            def t_body(t, cnt):
                o = (r * STEPS + t) * 16
                ix = ixb[pl.ds(o, 16)]
                th = thb[pl.ds(o, 16)]
                a = plsc.load_gather(sg_v, [ix])
                new = jnp.where(a >= th, 1.0, 0.0).astype(jnp.float32)
                old = plsc.load_gather(mg_v, [ix])
                plsc.store_scatter(mg_v, [ix], new)
                sg = new - old
                sgb[pl.ds(o, 16)] = sg
                return cnt + sg

            cnt = lax.fori_loop(0, STEPS, t_body, zero16)
            wr = 0.25 / jnp.maximum(cnt, 1.0)

            def t2_body(t, carry2):
                o = (r * STEPS + t) * 16
                ix = ixb[pl.ds(o, 16)]
                sg = sgb[pl.ds(o, 16)]
                plsc.addupdate_scatter(wg_v, [ix], sg * wr)
                plsc.store_scatter(mg_v, [ix], zero16)
                return carry2

            lax.fori_loop(0, STEPS, t2_body, 0)
        pltpu.sync_copy(wg_v, w_hbm.at[pl.ds(soff, ROWS)])

        def t3_body(i, carry3):
            ix = ixb[pl.ds(i * 16, 16)]
            plsc.store_scatter(wg_v, [ix], zero16)
            return carry3

        lax.fori_loop(0, RUNS * STEPS, t3_body, 0)
        return carry

    lax.fori_loop(0, NG, group_body, 0)


_sc_weights = functools.partial(
    pl.kernel,
    out_type=jax.ShapeDtypeStruct((NQ * L,), jnp.float32),
    mesh=plsc.VectorSubcoreMesh(core_axis_name="c", subcore_axis_name="s"),
    compiler_params=pltpu.CompilerParams(needs_layout_passes=False),
    scratch_types=[
        pltpu.VMEM((ROWS,), jnp.float32),
        pltpu.VMEM((ROWS,), jnp.float32),
        pltpu.VMEM((ROWS,), jnp.float32),
        pltpu.VMEM((GROUP,), jnp.int32),
        pltpu.VMEM((GROUP,), jnp.float32),
        pltpu.VMEM((GROUP,), jnp.float32),
    ],
)(_sc_body)


def kernel(q, k, v):
    qf = q.astype(jnp.float32)
    kf = k.astype(jnp.float32)
    vf = v.astype(jnp.float32)

    rkey = jax.random.key(1234)
    k1, k2 = jax.random.split(rkey)
    vidx_all = jax.random.randint(k1, (STEPS, NCH), 0, L)
    z_all = jax.random.uniform(k2, (STEPS, NCH), dtype=jnp.float32)
    th_all = (jnp.log(z_all) - jnp.log1p(-z_all)) / BETA

    vi = vidx_all.astype(jnp.int32).reshape(STEPS, NW, NG, QPG, RUNS)
    vi = vi.transpose(1, 2, 4, 0, 3)
    lane = jnp.arange(QPG, dtype=jnp.int32)
    ixg = (lane * L + vi).reshape(-1)
    thg = th_all.reshape(STEPS, NW, NG, QPG, RUNS).transpose(1, 2, 4, 0, 3).reshape(-1)

    s_mat = pl.pallas_call(
        _qk_body,
        grid=(B,),
        in_specs=[
            pl.BlockSpec((1, Lq, D), lambda b: (b, 0, 0)),
            pl.BlockSpec((1, D, L), lambda b: (b, 0, 0)),
        ],
        out_specs=pl.BlockSpec((1, Lq, L), lambda b: (b, 0, 0)),
        out_shape=jax.ShapeDtypeStruct((B, Lq, L), jnp.float32),
    )(qf, kf.transpose(0, 2, 1))

    w_flat = _sc_weights(s_mat.reshape(NQ * L), ixg, thg)

    bq = 512
    out = pl.pallas_call(
        _wv_body,
        grid=(B, Lq // bq),
        in_specs=[
            pl.BlockSpec((1, bq, L), lambda b, i: (b, i, 0)),
            pl.BlockSpec((1, L, D), lambda b, i: (b, 0, 0)),
        ],
        out_specs=pl.BlockSpec((1, bq, D), lambda b, i: (b, i, 0)),
        out_shape=jax.ShapeDtypeStruct((B, Lq, D), jnp.float32),
    )(w_flat.reshape(B, Lq, L), vf)
    return out


# speedup vs baseline: 307.3396x; 1.2854x over previous
import functools
import math

import jax
import jax.numpy as jnp
from jax import lax
from jax.experimental import pallas as pl
from jax.experimental.pallas import tpu as pltpu
from jax.experimental.pallas import tpu_sc as plsc

B, Lq, L, D = 2, 2048, 2048, 64
RUNS, STEPS = 4, 32
BETA = 1.0
SCALE = 1.0 / math.sqrt(D)
NQ = B * Lq
NCH = NQ * RUNS
NW = 32
QPW = NQ // NW
QPG = 16
NG = QPW // QPG
GROUP = RUNS * STEPS * QPG
ROWS = QPG * L


def _qk_body(q_ref, kt_ref, s_ref):
    s_ref[0] = lax.dot_general(
        q_ref[0], kt_ref[0], (((1,), (0,)), ((), ())),
        precision=lax.Precision.HIGHEST,
        preferred_element_type=jnp.float32) * SCALE


def _wv_body(w_ref, v_ref, o_ref):
    o_ref[0] = lax.dot_general(
        w_ref[0], v_ref[0], (((1,), (0,)), ((), ())),
        precision=lax.Precision.HIGHEST,
        preferred_element_type=jnp.float32)


def _sc_body(s_hbm, vi_hbm, th_hbm, w_hbm, sg_v, mg_v, wg_v, vb, tb, ixb, sgb):
    c = lax.axis_index("c")
    s = lax.axis_index("s")
    w = s * 2 + c
    zero16 = jnp.zeros((16,), jnp.float32)
    lane = lax.iota(jnp.int32, 16)
    lane_l = lane * L

    def zinit(i, carry):
        mg_v[pl.ds(i * 16, 16)] = zero16
        wg_v[pl.ds(i * 16, 16)] = zero16
        return carry

    lax.fori_loop(0, ROWS // 16, zinit, 0)

    def group_body(g, carry):
        wg = w * NG + g
        soff = wg * ROWS
        pltpu.sync_copy(s_hbm.at[pl.ds(soff, ROWS)], sg_v)
        pltpu.sync_copy(vi_hbm.at[:, pl.ds((wg >> 1) * 128, 128)], vb)
        pltpu.sync_copy(th_hbm.at[:, pl.ds((wg >> 1) * 128, 128)], tb)
        half = (wg & 1) * 64
        for r in range(RUNS):
            lane_r = lane * RUNS + r + half

            def t_body(t, cnt):
                tvec = jnp.full((16,), t, jnp.int32)
                vi = plsc.load_gather(vb, [tvec, lane_r])
                th = plsc.load_gather(tb, [tvec, lane_r])
                ix = lane_l + vi
                a = plsc.load_gather(sg_v, [ix])
                new = jnp.where(a >= th, 1.0, 0.0).astype(jnp.float32)
                old = plsc.load_gather(mg_v, [ix])
                plsc.store_scatter(mg_v, [ix], new)
                sg = new - old
                o = (r * STEPS + t) * 16
                ixb[pl.ds(o, 16)] = ix
                sgb[pl.ds(o, 16)] = sg
                return cnt + sg

            cnt = lax.fori_loop(0, STEPS, t_body, zero16)
            wr = 0.25 / jnp.maximum(cnt, 1.0)

            def t2_body(t, carry2):
                o = (r * STEPS + t) * 16
                ix = ixb[pl.ds(o, 16)]
                sg = sgb[pl.ds(o, 16)]
                plsc.addupdate_scatter(wg_v, [ix], sg * wr)
                plsc.store_scatter(mg_v, [ix], zero16)
                return carry2

            lax.fori_loop(0, STEPS, t2_body, 0)
        pltpu.sync_copy(wg_v, w_hbm.at[pl.ds(soff, ROWS)])

        def t3_body(i, carry3):
            ix = ixb[pl.ds(i * 16, 16)]
            plsc.store_scatter(wg_v, [ix], zero16)
            return carry3

        lax.fori_loop(0, RUNS * STEPS, t3_body, 0)
        return carry

    lax.fori_loop(0, NG, group_body, 0)


_sc_weights = functools.partial(
    pl.kernel,
    out_type=jax.ShapeDtypeStruct((NQ * L,), jnp.float32),
    mesh=plsc.VectorSubcoreMesh(core_axis_name="c", subcore_axis_name="s"),
    compiler_params=pltpu.CompilerParams(needs_layout_passes=False),
    scratch_types=[
        pltpu.VMEM((ROWS,), jnp.float32),
        pltpu.VMEM((ROWS,), jnp.float32),
        pltpu.VMEM((ROWS,), jnp.float32),
        pltpu.VMEM((STEPS, 2 * RUNS * QPG), jnp.int32),
        pltpu.VMEM((STEPS, 2 * RUNS * QPG), jnp.float32),
        pltpu.VMEM((GROUP,), jnp.int32),
        pltpu.VMEM((GROUP,), jnp.float32),
    ],
)(_sc_body)


def kernel(q, k, v):
    qf = q.astype(jnp.float32)
    kf = k.astype(jnp.float32)
    vf = v.astype(jnp.float32)

    rkey = jax.random.key(1234)
    k1, k2 = jax.random.split(rkey)
    vidx_all = jax.random.randint(k1, (STEPS, NCH), 0, L)
    z_all = jax.random.uniform(k2, (STEPS, NCH), dtype=jnp.float32)
    th_all = (jnp.log(z_all) - jnp.log1p(-z_all)) / BETA
    vidx_all = vidx_all.astype(jnp.int32)

    s_mat = pl.pallas_call(
        _qk_body,
        grid=(B,),
        in_specs=[
            pl.BlockSpec((1, Lq, D), lambda b: (b, 0, 0)),
            pl.BlockSpec((1, D, L), lambda b: (b, 0, 0)),
        ],
        out_specs=pl.BlockSpec((1, Lq, L), lambda b: (b, 0, 0)),
        out_shape=jax.ShapeDtypeStruct((B, Lq, L), jnp.float32),
    )(qf, kf.transpose(0, 2, 1))

    w_flat = _sc_weights(s_mat.reshape(NQ * L), vidx_all, th_all)

    bq = 512
    out = pl.pallas_call(
        _wv_body,
        grid=(B, Lq // bq),
        in_specs=[
            pl.BlockSpec((1, bq, L), lambda b, i: (b, i, 0)),
            pl.BlockSpec((1, L, D), lambda b, i: (b, 0, 0)),
        ],
        out_specs=pl.BlockSpec((1, bq, D), lambda b, i: (b, i, 0)),
        out_shape=jax.ShapeDtypeStruct((B, Lq, D), jnp.float32),
    )(w_flat.reshape(B, Lq, L), vf)
    return out


# speedup vs baseline: 357.7846x; 1.1641x over previous
import functools
import math

import jax
import jax.numpy as jnp
from jax import lax
from jax.experimental import pallas as pl
from jax.experimental.pallas import tpu as pltpu
from jax.experimental.pallas import tpu_sc as plsc

B, Lq, L, D = 2, 2048, 2048, 64
RUNS, STEPS = 4, 32
BETA = 1.0
SCALE = 1.0 / math.sqrt(D)
NQ = B * Lq
NCH = NQ * RUNS
NW = 32
QPW = NQ // NW
QPG = 16
NG = QPW // QPG
GROUP = RUNS * STEPS * QPG
ROWS = QPG * L


def _x3_matmul(a, b):
    ah = a.astype(jnp.bfloat16)
    al = (a - ah.astype(jnp.float32)).astype(jnp.bfloat16)
    bh = b.astype(jnp.bfloat16)
    bl = (b - bh.astype(jnp.float32)).astype(jnp.bfloat16)
    dn = (((1,), (0,)), ((), ()))
    acc = lax.dot_general(ah, bh, dn, preferred_element_type=jnp.float32)
    acc += lax.dot_general(ah, bl, dn, preferred_element_type=jnp.float32)
    acc += lax.dot_general(al, bh, dn, preferred_element_type=jnp.float32)
    return acc


def _qk_body(q_ref, kt_ref, s_ref):
    s_ref[0] = _x3_matmul(q_ref[0], kt_ref[0]) * SCALE


def _wv_body(w_ref, v_ref, o_ref):
    o_ref[0] = _x3_matmul(w_ref[0], v_ref[0])


def _sc_body(s_hbm, vi_hbm, th_hbm, w_hbm, sg_v, mg_v, wg_v, vb, tb, ixb, sgb,
             sem):
    c = lax.axis_index("c")
    s = lax.axis_index("s")
    w = s * 2 + c
    zero16 = jnp.zeros((16,), jnp.float32)
    lane = lax.iota(jnp.int32, 16)
    lane_l = lane * L

    def zinit(i, carry):
        mg_v[pl.ds(i * 16, 16)] = zero16
        wg_v[pl.ds(i * 16, 16)] = zero16
        return carry

    lax.fori_loop(0, ROWS // 16, zinit, 0)

    def group_body(g, carry):
        wg = w * NG + g
        soff = wg * ROWS
        cp_s = pltpu.async_copy(s_hbm.at[pl.ds(soff, ROWS)], sg_v, sem)
        cp_v = pltpu.async_copy(
            vi_hbm.at[:, pl.ds((wg >> 1) * 128, 128)], vb, sem)
        cp_t = pltpu.async_copy(
            th_hbm.at[:, pl.ds((wg >> 1) * 128, 128)], tb, sem)
        cp_s.wait()
        cp_v.wait()
        cp_t.wait()
        half = (wg & 1) * 64
        for r in range(RUNS):
            lane_r = lane * RUNS + r + half

            def step1(t, cnt):
                tvec = jnp.full((16,), t, jnp.int32)
                vi = plsc.load_gather(vb, [tvec, lane_r])
                th = plsc.load_gather(tb, [tvec, lane_r])
                ix = lane_l + vi
                a = plsc.load_gather(sg_v, [ix])
                new = jnp.where(a >= th, 1.0, 0.0).astype(jnp.float32)
                old = plsc.load_gather(mg_v, [ix])
                plsc.store_scatter(mg_v, [ix], new)
                sg = new - old
                o = (r * STEPS + t) * 16
                ixb[pl.ds(o, 16)] = ix
                sgb[pl.ds(o, 16)] = sg
                return cnt + sg

            def t_body(t4, cnt):
                for u in range(4):
                    cnt = step1(t4 * 4 + u, cnt)
                return cnt

            cnt = lax.fori_loop(0, STEPS // 4, t_body, zero16)
            wr = 0.25 / jnp.maximum(cnt, 1.0)

            def t2_body(t4, carry2):
                for u in range(4):
                    o = (r * STEPS + t4 * 4 + u) * 16
                    ix = ixb[pl.ds(o, 16)]
                    sg = sgb[pl.ds(o, 16)]
                    plsc.addupdate_scatter(wg_v, [ix], sg * wr)
                    plsc.store_scatter(mg_v, [ix], zero16)
                return carry2

            lax.fori_loop(0, STEPS // 4, t2_body, 0)
        pltpu.sync_copy(wg_v, w_hbm.at[pl.ds(soff, ROWS)])

        def t3_body(i4, carry3):
            for u in range(4):
                ix = ixb[pl.ds((i4 * 4 + u) * 16, 16)]
                plsc.store_scatter(wg_v, [ix], zero16)
            return carry3

        lax.fori_loop(0, RUNS * STEPS // 4, t3_body, 0)
        return carry

    lax.fori_loop(0, NG, group_body, 0)


_sc_weights = functools.partial(
    pl.kernel,
    out_type=jax.ShapeDtypeStruct((NQ * L,), jnp.float32),
    mesh=plsc.VectorSubcoreMesh(core_axis_name="c", subcore_axis_name="s"),
    compiler_params=pltpu.CompilerParams(needs_layout_passes=False),
    scratch_types=[
        pltpu.VMEM((ROWS,), jnp.float32),
        pltpu.VMEM((ROWS,), jnp.float32),
        pltpu.VMEM((ROWS,), jnp.float32),
        pltpu.VMEM((STEPS, 2 * RUNS * QPG), jnp.int32),
        pltpu.VMEM((STEPS, 2 * RUNS * QPG), jnp.float32),
        pltpu.VMEM((GROUP,), jnp.int32),
        pltpu.VMEM((GROUP,), jnp.float32),
        pltpu.SemaphoreType.DMA,
    ],
)(_sc_body)


def kernel(q, k, v):
    qf = q.astype(jnp.float32)
    kf = k.astype(jnp.float32)
    vf = v.astype(jnp.float32)

    rkey = jax.random.key(1234)
    k1, k2 = jax.random.split(rkey)
    vidx_all = jax.random.randint(k1, (STEPS, NCH), 0, L)
    z_all = jax.random.uniform(k2, (STEPS, NCH), dtype=jnp.float32)
    th_all = (jnp.log(z_all) - jnp.log1p(-z_all)) / BETA
    vidx_all = vidx_all.astype(jnp.int32)

    s_mat = pl.pallas_call(
        _qk_body,
        grid=(B,),
        in_specs=[
            pl.BlockSpec((1, Lq, D), lambda b: (b, 0, 0)),
            pl.BlockSpec((1, D, L), lambda b: (b, 0, 0)),
        ],
        out_specs=pl.BlockSpec((1, Lq, L), lambda b: (b, 0, 0)),
        out_shape=jax.ShapeDtypeStruct((B, Lq, L), jnp.float32),
    )(qf, kf.transpose(0, 2, 1))

    w_flat = _sc_weights(s_mat.reshape(NQ * L), vidx_all, th_all)

    bq = 512
    out = pl.pallas_call(
        _wv_body,
        grid=(B, Lq // bq),
        in_specs=[
            pl.BlockSpec((1, bq, L), lambda b, i: (b, i, 0)),
            pl.BlockSpec((1, L, D), lambda b, i: (b, 0, 0)),
        ],
        out_specs=pl.BlockSpec((1, bq, D), lambda b, i: (b, i, 0)),
        out_shape=jax.ShapeDtypeStruct((B, Lq, D), jnp.float32),
    )(w_flat.reshape(B, Lq, L), vf)
    return out
